# baseline (speedup 1.0000x reference)
import functools

import jax
import jax.numpy as jnp
from jax import lax
from jax.experimental import pallas as pl
from jax.experimental.pallas import tpu as pltpu
from jax.experimental.pallas import tpu_sc as plsc

_B, _S, _V = 8, 2048, 4096
_S_BLK = 1024
_NSB = _S // _S_BLK
_L = 16
_NC = 2
_NT = 16
_N_SC_BATCH = 2
_N_TC_BATCH = _B - _N_SC_BATCH
_ROWS_PER_TILE = _S // _NT
_ROW_CHUNK = 16
_MAX_COPIES = 4.0
_NUM_SPECIAL = 3
_BIG = jnp.int32(1 << 30)


def _argmax_body(x_ref, tok_ref):
    x = x_ref[0]
    m = jnp.max(x, axis=1, keepdims=True)
    iota = lax.broadcasted_iota(jnp.int32, (_S_BLK, _V), 1)
    idx = jnp.min(jnp.where(x == m, iota, _V), axis=1, keepdims=True)
    tok_ref[0] = idx.reshape(1, _S_BLK)


def _sc_argmax_body(xflat_hbm, tokflat_hbm, x_v, tok_v):
    c = lax.axis_index("c")
    s = lax.axis_index("s")

    @pl.when(c < _N_SC_BATCH)
    def _():
        b = _N_TC_BATCH + c
        lane = lax.iota(jnp.int32, _L)
        n_chunks = _ROWS_PER_TILE // _ROW_CHUNK

        def chunk_body(ch, carry):
            row0 = s * _ROWS_PER_TILE + ch * _ROW_CHUNK
            pltpu.sync_copy(xflat_hbm.at[pl.ds(b * _S + row0, _ROW_CHUNK)], x_v)
            acc = jnp.zeros((_L,), jnp.int32)
            for r in range(_ROW_CHUNK):
                def p1(ci, m16):
                    return jnp.maximum(m16, x_v[r, pl.ds(ci * _L, _L)])

                m16 = lax.fori_loop(
                    0, _V // _L, p1, jnp.full((_L,), -jnp.inf, jnp.float32),
                    unroll=8,
                )

                def p2(ci, cidx):
                    eq = x_v[r, pl.ds(ci * _L, _L)] == m16
                    return jnp.minimum(cidx, jnp.where(eq, ci, _BIG))

                cidx = lax.fori_loop(
                    0, _V // _L, p2, jnp.full((_L,), _BIG, jnp.int32),
                    unroll=8,
                )
                rowmax = jnp.max(m16)
                gidx = jnp.min(
                    jnp.where(m16 == rowmax, cidx * _L + lane, _BIG)
                )
                acc = jnp.where(lane == r, gidx, acc)
            tok_v[pl.ds(ch * _L, _L)] = acc
            return carry

        lax.fori_loop(0, n_chunks, chunk_body, 0)
        base = c * _S + s * _ROWS_PER_TILE
        pltpu.sync_copy(tok_v, tokflat_hbm.at[pl.ds(base, _ROWS_PER_TILE)])


def _hist_body(tok_hbm, zeros_hbm, out_hbm, tok_v, hist_v):
    c = lax.axis_index("c")
    s = lax.axis_index("s")
    wid = s * _NC + c

    @pl.when(wid < _B)
    def _():
        pltpu.sync_copy(zeros_hbm, hist_v)
        pltpu.sync_copy(tok_hbm.at[wid], tok_v)

        ones = jnp.ones((_L,), jnp.float32)

        def scat(i, carry):
            idx = tok_v[pl.ds(i * _L, _L)]
            plsc.addupdate_scatter(hist_v, [idx], ones)
            return carry

        lax.fori_loop(0, _S // _L, scat, 0)

        def fin(j, carry):
            v = hist_v[pl.ds(j * _L, _L)]
            pos = lax.iota(jnp.int32, _L) + j * _L
            v = jnp.where(
                pos >= _NUM_SPECIAL,
                jnp.minimum(v, _MAX_COPIES) * (1.0 / _MAX_COPIES),
                0.0,
            )
            hist_v[pl.ds(j * _L, _L)] = v
            return carry

        lax.fori_loop(0, _V // _L, fin, 0)
        pltpu.sync_copy(hist_v, out_hbm.at[wid])


def kernel(main_logits):
    tok_raw = pl.pallas_call(
        _argmax_body,
        grid=(_N_TC_BATCH, _NSB),
        in_specs=[pl.BlockSpec((1, _S_BLK, _V), lambda b, sb: (b, sb, 0))],
        out_specs=pl.BlockSpec((1, 1, _S_BLK), lambda b, sb: (b * _NSB + sb, 0, 0)),
        out_shape=jax.ShapeDtypeStruct((_N_TC_BATCH * _NSB, 1, _S_BLK), jnp.int32),
        compiler_params=pltpu.CompilerParams(
            dimension_semantics=("arbitrary", "arbitrary")
        ),
    )(main_logits)
    tok_tc = tok_raw.reshape(_N_TC_BATCH, _S)

    xflat = main_logits.reshape(_B * _S, _V)
    tok_sc_flat = pl.kernel(
        _sc_argmax_body,
        mesh=plsc.VectorSubcoreMesh(core_axis_name="c", subcore_axis_name="s"),
        out_type=jax.ShapeDtypeStruct((_N_SC_BATCH * _S,), jnp.int32),
        scratch_types=[
            pltpu.VMEM((_ROW_CHUNK, _V), jnp.float32),
            pltpu.VMEM((_ROWS_PER_TILE,), jnp.int32),
        ],
        compiler_params=pltpu.CompilerParams(needs_layout_passes=False),
    )(xflat)
    tok_sc = tok_sc_flat.reshape(_N_SC_BATCH, _S)

    tokens = jnp.concatenate([tok_tc, tok_sc], axis=0)
    zeros = jnp.zeros((_V,), jnp.float32)

    hist = pl.kernel(
        _hist_body,
        mesh=plsc.VectorSubcoreMesh(core_axis_name="c", subcore_axis_name="s"),
        out_type=jax.ShapeDtypeStruct((_B, _V), jnp.float32),
        scratch_types=[
            pltpu.VMEM((_S,), jnp.int32),
            pltpu.VMEM((_V,), jnp.float32),
        ],
        compiler_params=pltpu.CompilerParams(needs_layout_passes=False),
    )(tokens, zeros)
    return hist


# speedup vs baseline: 1.1539x; 1.1539x over previous
import functools

import jax
import jax.numpy as jnp
from jax import lax
from jax.experimental import pallas as pl
from jax.experimental.pallas import tpu as pltpu
from jax.experimental.pallas import tpu_sc as plsc

_B, _S, _V = 8, 2048, 4096
_S_BLK = 1024
_NSB = _S // _S_BLK
_L = 16
_NC = 2
_NT = 16
_N_SC_BATCH = 2
_N_TC_BATCH = _B - _N_SC_BATCH
_ROWS_PER_TILE = _S // _NT
_ROW_CHUNK = 16
_MAX_COPIES = 4.0
_NUM_SPECIAL = 3
_BIG = jnp.int32(1 << 30)


def _argmax_body(x_ref, tok_ref):
    x = x_ref[0]
    m = jnp.max(x, axis=1, keepdims=True)
    iota = lax.broadcasted_iota(jnp.int32, (_S_BLK, _V), 1)
    idx = jnp.min(jnp.where(x == m, iota, _V), axis=1, keepdims=True)
    tok_ref[0] = idx.reshape(1, _S_BLK)


def _sc_argmax_body(xflat_hbm, tokflat_hbm, x_v, tok_v):
    c = lax.axis_index("c")
    s = lax.axis_index("s")

    @pl.when(c < _N_SC_BATCH)
    def _():
        b = _N_TC_BATCH + c
        lane = lax.iota(jnp.int32, _L)
        n_chunks = _ROWS_PER_TILE // _ROW_CHUNK

        def chunk_body(ch, carry):
            row0 = s * _ROWS_PER_TILE + ch * _ROW_CHUNK
            pltpu.sync_copy(xflat_hbm.at[pl.ds(b * _S + row0, _ROW_CHUNK)], x_v)
            acc = jnp.zeros((_L,), jnp.int32)
            for r in range(_ROW_CHUNK):
                def p1(ci, carry):
                    m16, cidx = carry
                    x = x_v[r, pl.ds(ci * _L, _L)]
                    gt = x > m16
                    return jnp.maximum(x, m16), jnp.where(gt, ci, cidx)

                m16, cidx = lax.fori_loop(
                    0, _V // _L, p1,
                    (jnp.full((_L,), -jnp.inf, jnp.float32),
                     jnp.full((_L,), _BIG, jnp.int32)),
                    unroll=8,
                )
                rowmax = jnp.max(m16)
                gidx = jnp.min(
                    jnp.where(m16 == rowmax, cidx * _L + lane, _BIG)
                )
                acc = jnp.where(lane == r, gidx, acc)
            tok_v[pl.ds(ch * _L, _L)] = acc
            return carry

        lax.fori_loop(0, n_chunks, chunk_body, 0)
        base = c * _S + s * _ROWS_PER_TILE
        pltpu.sync_copy(tok_v, tokflat_hbm.at[pl.ds(base, _ROWS_PER_TILE)])


def _hist_body(tok_hbm, zeros_hbm, out_hbm, tok_v, hist_v):
    c = lax.axis_index("c")
    s = lax.axis_index("s")
    wid = s * _NC + c

    @pl.when(wid < _B)
    def _():
        pltpu.sync_copy(zeros_hbm, hist_v)
        pltpu.sync_copy(tok_hbm.at[wid], tok_v)

        ones = jnp.ones((_L,), jnp.float32)

        def scat(i, carry):
            idx = tok_v[pl.ds(i * _L, _L)]
            plsc.addupdate_scatter(hist_v, [idx], ones)
            return carry

        lax.fori_loop(0, _S // _L, scat, 0)

        def fin(j, carry):
            v = hist_v[pl.ds(j * _L, _L)]
            pos = lax.iota(jnp.int32, _L) + j * _L
            v = jnp.where(
                pos >= _NUM_SPECIAL,
                jnp.minimum(v, _MAX_COPIES) * (1.0 / _MAX_COPIES),
                0.0,
            )
            hist_v[pl.ds(j * _L, _L)] = v
            return carry

        lax.fori_loop(0, _V // _L, fin, 0)
        pltpu.sync_copy(hist_v, out_hbm.at[wid])


def kernel(main_logits):
    tok_raw = pl.pallas_call(
        _argmax_body,
        grid=(_N_TC_BATCH, _NSB),
        in_specs=[pl.BlockSpec((1, _S_BLK, _V), lambda b, sb: (b, sb, 0))],
        out_specs=pl.BlockSpec((1, 1, _S_BLK), lambda b, sb: (b * _NSB + sb, 0, 0)),
        out_shape=jax.ShapeDtypeStruct((_N_TC_BATCH * _NSB, 1, _S_BLK), jnp.int32),
        compiler_params=pltpu.CompilerParams(
            dimension_semantics=("arbitrary", "arbitrary")
        ),
    )(main_logits)
    tok_tc = tok_raw.reshape(_N_TC_BATCH, _S)

    xflat = main_logits.reshape(_B * _S, _V)
    tok_sc_flat = pl.kernel(
        _sc_argmax_body,
        mesh=plsc.VectorSubcoreMesh(core_axis_name="c", subcore_axis_name="s"),
        out_type=jax.ShapeDtypeStruct((_N_SC_BATCH * _S,), jnp.int32),
        scratch_types=[
            pltpu.VMEM((_ROW_CHUNK, _V), jnp.float32),
            pltpu.VMEM((_ROWS_PER_TILE,), jnp.int32),
        ],
        compiler_params=pltpu.CompilerParams(needs_layout_passes=False),
    )(xflat)
    tok_sc = tok_sc_flat.reshape(_N_SC_BATCH, _S)

    tokens = jnp.concatenate([tok_tc, tok_sc], axis=0)
    zeros = jnp.zeros((_V,), jnp.float32)

    hist = pl.kernel(
        _hist_body,
        mesh=plsc.VectorSubcoreMesh(core_axis_name="c", subcore_axis_name="s"),
        out_type=jax.ShapeDtypeStruct((_B, _V), jnp.float32),
        scratch_types=[
            pltpu.VMEM((_S,), jnp.int32),
            pltpu.VMEM((_V,), jnp.float32),
        ],
        compiler_params=pltpu.CompilerParams(needs_layout_passes=False),
    )(tokens, zeros)
    return hist


# speedup vs baseline: 1.2250x; 1.0617x over previous
import functools

import jax
import jax.numpy as jnp
from jax import lax
from jax.experimental import pallas as pl
from jax.experimental.pallas import tpu as pltpu
from jax.experimental.pallas import tpu_sc as plsc

_B, _S, _V = 8, 2048, 4096
_S_BLK = 1024
_NSB = _S // _S_BLK
_L = 16
_NC = 2
_MAX_COPIES = 4.0
_NUM_SPECIAL = 3


def _argmax_body(x_ref, tok_ref):
    x = x_ref[0]
    m = jnp.max(x, axis=1, keepdims=True)
    iota = lax.broadcasted_iota(jnp.int32, (_S_BLK, _V), 1)
    idx = jnp.min(jnp.where(x == m, iota, _V), axis=1, keepdims=True)
    tok_ref[0] = idx.reshape(1, _S_BLK)


def _hist_body(tok_hbm, zeros_hbm, out_hbm, tok_v, hist_v):
    c = lax.axis_index("c")
    s = lax.axis_index("s")
    wid = s * _NC + c

    @pl.when(wid < _B)
    def _():
        pltpu.sync_copy(zeros_hbm, hist_v)
        pltpu.sync_copy(tok_hbm.at[wid], tok_v)

        ones = jnp.ones((_L,), jnp.float32)

        def scat(i, carry):
            idx = tok_v[pl.ds(i * _L, _L)]
            plsc.addupdate_scatter(hist_v, [idx], ones)
            return carry

        lax.fori_loop(0, _S // _L, scat, 0, unroll=8)

        def fin(j, carry):
            v = hist_v[pl.ds(j * _L, _L)]
            pos = lax.iota(jnp.int32, _L) + j * _L
            v = jnp.where(
                pos >= _NUM_SPECIAL,
                jnp.minimum(v, _MAX_COPIES) * (1.0 / _MAX_COPIES),
                0.0,
            )
            hist_v[pl.ds(j * _L, _L)] = v
            return carry

        lax.fori_loop(0, _V // _L, fin, 0, unroll=8)
        pltpu.sync_copy(hist_v, out_hbm.at[wid])


def kernel(main_logits):
    tok_raw = pl.pallas_call(
        _argmax_body,
        grid=(_B, _NSB),
        in_specs=[pl.BlockSpec((1, _S_BLK, _V), lambda b, sb: (b, sb, 0))],
        out_specs=pl.BlockSpec((1, 1, _S_BLK), lambda b, sb: (b * _NSB + sb, 0, 0)),
        out_shape=jax.ShapeDtypeStruct((_B * _NSB, 1, _S_BLK), jnp.int32),
        compiler_params=pltpu.CompilerParams(
            dimension_semantics=("arbitrary", "arbitrary")
        ),
    )(main_logits)
    tokens = tok_raw.reshape(_B, _S)
    zeros = jnp.zeros((_V,), jnp.float32)

    hist = pl.kernel(
        _hist_body,
        mesh=plsc.VectorSubcoreMesh(core_axis_name="c", subcore_axis_name="s"),
        out_type=jax.ShapeDtypeStruct((_B, _V), jnp.float32),
        scratch_types=[
            pltpu.VMEM((_S,), jnp.int32),
            pltpu.VMEM((_V,), jnp.float32),
        ],
        compiler_params=pltpu.CompilerParams(needs_layout_passes=False),
    )(tokens, zeros)
    return hist
